# baseline (speedup 1.0000x reference)
import functools

import jax
import jax.numpy as jnp
from jax import lax
from jax.experimental import pallas as pl
from jax.experimental.pallas import tpu as pltpu
from jax.experimental.pallas import tpu_sc as plsc

_B = 128
_S = 8192
_N = _B * _S
_NW = 32
_PER_W = _N // _NW
_C = 8192
_NCHUNK = _PER_W // _C

_LN2 = 0.6931471805599453
_SQRT2 = 1.4142135623730951


def _neg_ln(x):
    bits = lax.bitcast_convert_type(x, jnp.int32)
    e = lax.shift_right_logical(bits, 23) - 127
    m_bits = lax.bitwise_or(lax.bitwise_and(bits, 0x007FFFFF), 0x3F800000)
    m = lax.bitcast_convert_type(m_bits, jnp.float32)
    big = m > _SQRT2
    m = jnp.where(big, m * 0.5, m)
    ef = e.astype(jnp.float32) + jnp.where(big, 1.0, 0.0)
    s = (m - 1.0) / (m + 1.0)
    z = s * s
    p = 2.0 + z * (0.66666667 + z * (0.4 + z * 0.28571429))
    return -(ef * _LN2 + s * p)


def _sc_body(l_hbm, ua_hbm, ub_hbm, out_hbm, lv, av, bv, ov):
    wid = lax.axis_index("s") * 2 + lax.axis_index("c")
    base = wid * _PER_W
    for c in range(_NCHUNK):
        off = base + c * _C
        pltpu.sync_copy(l_hbm.at[pl.ds(off, _C)], lv)
        pltpu.sync_copy(ua_hbm.at[pl.ds(off, _C)], av)
        pltpu.sync_copy(ub_hbm.at[pl.ds(off, _C)], bv)

        def body(i, _):
            ix = pl.ds(i * 16, 16)
            la = _neg_ln(av[ix])
            lb = _neg_ln(bv[ix])
            t = jnp.exp(2.0 - 4.0 * lv[ix])
            bb = lb * lb
            ov[ix] = bb / (la * la * t + bb)
            return 0

        lax.fori_loop(0, _C // 16, body, 0)
        pltpu.sync_copy(ov, out_hbm.at[pl.ds(off, _C)])


@functools.cache
def _sc_call():
    return pl.kernel(
        _sc_body,
        out_type=jax.ShapeDtypeStruct((_N,), jnp.float32),
        mesh=plsc.VectorSubcoreMesh(core_axis_name="c", subcore_axis_name="s"),
        scratch_types=[
            pltpu.VMEM((_C,), jnp.float32),
            pltpu.VMEM((_C,), jnp.float32),
            pltpu.VMEM((_C,), jnp.float32),
            pltpu.VMEM((_C,), jnp.float32),
        ],
    )


@jax.jit
def kernel(logits, uniform_a, uniform_b):
    l = logits.reshape(_N)
    ua = uniform_a.reshape(_N)
    ub = uniform_b.reshape(_N)
    out = _sc_call()(l, ua, ub)
    return out.reshape(_B, _S, 1)


# speedup vs baseline: 1.0893x; 1.0893x over previous
import functools

import jax
import jax.numpy as jnp
from jax import lax
from jax.experimental import pallas as pl
from jax.experimental.pallas import tpu as pltpu
from jax.experimental.pallas import tpu_sc as plsc

_B = 128
_S = 8192
_N = _B * _S
_NW = 32
_PER_W = _N // _NW
_C = 8192
_NCHUNK = _PER_W // _C

_C0 = 2.885390081777927
_C1 = 0.961796693925976
_C2 = 0.5770780163555854
_C3 = 0.41219858311113243
_C4 = 0.3205988979753253
_C5 = 0.26230818925254066


def _neg_log2(x, inv, den_other):
    bits = lax.bitcast_convert_type(x, jnp.int32)
    ke = 127 - lax.shift_right_logical(bits, 23)
    m_bits = lax.bitwise_or(lax.bitwise_and(bits, 0x007FFFFF), 0x3F800000)
    m = lax.bitcast_convert_type(m_bits, jnp.float32)
    s = (m - 1.0) * (den_other * inv)
    z = s * s
    p = _C0 + z * (_C1 + z * (_C2 + z * (_C3 + z * (_C4 + z * _C5))))
    return ke.astype(jnp.float32) - s * p


def _mant_p1(x):
    bits = lax.bitcast_convert_type(x, jnp.int32)
    m_bits = lax.bitwise_or(lax.bitwise_and(bits, 0x007FFFFF), 0x3F800000)
    return lax.bitcast_convert_type(m_bits, jnp.float32) + 1.0


def _sample(l, a, b):
    den_a = _mant_p1(a)
    den_b = _mant_p1(b)
    inv = 1.0 / (den_a * den_b)
    ka = _neg_log2(a, inv, den_b)
    kb = _neg_log2(b, inv, den_a)
    t = jnp.exp(2.0 - 4.0 * l)
    bb = kb * kb
    return bb / (ka * ka * t + bb)


def _sc_body(l_hbm, ua_hbm, ub_hbm, out_hbm,
             lv, av, bv, ov, isem0, isem1, osem0, osem1):
    wid = lax.axis_index("s") * 2 + lax.axis_index("c")
    base = wid * _PER_W
    isems = (isem0, isem1)
    osems = (osem0, osem1)

    def start_in(c):
        p = c % 2
        off = base + c * _C
        return [
            pltpu.async_copy(l_hbm.at[pl.ds(off, _C)], lv.at[p], isems[p]),
            pltpu.async_copy(ua_hbm.at[pl.ds(off, _C)], av.at[p], isems[p]),
            pltpu.async_copy(ub_hbm.at[pl.ds(off, _C)], bv.at[p], isems[p]),
        ]

    in_h = {0: start_in(0)}
    out_h = {}
    for c in range(_NCHUNK):
        p = c % 2
        if c + 1 < _NCHUNK:
            in_h[c + 1] = start_in(c + 1)
        for h in in_h.pop(c):
            h.wait()
        if c - 2 in out_h:
            out_h.pop(c - 2).wait()

        @plsc.parallel_loop(0, _C, step=16, unroll=8)
        def body(i):
            ix = pl.ds(i, 16)
            ov[p, ix] = _sample(lv[p, ix], av[p, ix], bv[p, ix])

        out_h[c] = pltpu.async_copy(
            ov.at[p], out_hbm.at[pl.ds(base + c * _C, _C)], osems[p]
        )
    for c in sorted(out_h):
        out_h.pop(c).wait()


@functools.cache
def _sc_call():
    return pl.kernel(
        _sc_body,
        out_type=jax.ShapeDtypeStruct((_N,), jnp.float32),
        mesh=plsc.VectorSubcoreMesh(core_axis_name="c", subcore_axis_name="s"),
        scratch_types=[
            pltpu.VMEM((2, _C), jnp.float32),
            pltpu.VMEM((2, _C), jnp.float32),
            pltpu.VMEM((2, _C), jnp.float32),
            pltpu.VMEM((2, _C), jnp.float32),
            pltpu.SemaphoreType.DMA,
            pltpu.SemaphoreType.DMA,
            pltpu.SemaphoreType.DMA,
            pltpu.SemaphoreType.DMA,
        ],
    )


@jax.jit
def kernel(logits, uniform_a, uniform_b):
    l = logits.reshape(_N)
    ua = uniform_a.reshape(_N)
    ub = uniform_b.reshape(_N)
    out = _sc_call()(l, ua, ub)
    return out.reshape(_B, _S, 1)


# speedup vs baseline: 1.1931x; 1.0953x over previous
import functools

import jax
import jax.numpy as jnp
from jax import lax
from jax.experimental import pallas as pl
from jax.experimental.pallas import tpu as pltpu
from jax.experimental.pallas import tpu_sc as plsc

_B = 128
_S = 8192
_N = _B * _S
_NW = 32
_PER_W = _N // _NW
_C = 8192
_NCHUNK = _PER_W // _C

_C0 = 2.88538788
_C1 = 0.9620558
_C2 = 0.56891856
_C3 = 0.5052695


def _neg_log2(x, inv, den_other):
    bits = lax.bitcast_convert_type(x, jnp.int32)
    ke = 127 - lax.shift_right_logical(bits, 23)
    m_bits = lax.bitwise_or(lax.bitwise_and(bits, 0x007FFFFF), 0x3F800000)
    m = lax.bitcast_convert_type(m_bits, jnp.float32)
    s = (m - 1.0) * (den_other * inv)
    z = s * s
    p = _C0 + z * (_C1 + z * (_C2 + z * _C3))
    return ke.astype(jnp.float32) - s * p


def _mant_p1(x):
    bits = lax.bitcast_convert_type(x, jnp.int32)
    m_bits = lax.bitwise_or(lax.bitwise_and(bits, 0x007FFFFF), 0x3F800000)
    return lax.bitcast_convert_type(m_bits, jnp.float32) + 1.0


def _sample(l, a, b):
    den_a = _mant_p1(a)
    den_b = _mant_p1(b)
    inv = 1.0 / (den_a * den_b)
    ka = _neg_log2(a, inv, den_b)
    kb = _neg_log2(b, inv, den_a)
    t = jnp.exp(2.0 - 4.0 * l)
    bb = kb * kb
    return bb / (ka * ka * t + bb)


def _sc_body(l_hbm, ua_hbm, ub_hbm, out_hbm,
             lv, av, bv, ov, isem0, isem1, osem0, osem1):
    wid = lax.axis_index("s") * 2 + lax.axis_index("c")
    base = wid * _PER_W
    isems = (isem0, isem1)
    osems = (osem0, osem1)

    def start_in(c):
        p = c % 2
        off = base + c * _C
        return [
            pltpu.async_copy(l_hbm.at[pl.ds(off, _C)], lv.at[p], isems[p]),
            pltpu.async_copy(ua_hbm.at[pl.ds(off, _C)], av.at[p], isems[p]),
            pltpu.async_copy(ub_hbm.at[pl.ds(off, _C)], bv.at[p], isems[p]),
        ]

    in_h = {0: start_in(0)}
    out_h = {}
    for c in range(_NCHUNK):
        p = c % 2
        if c + 1 < _NCHUNK:
            in_h[c + 1] = start_in(c + 1)
        for h in in_h.pop(c):
            h.wait()
        if c - 2 in out_h:
            out_h.pop(c - 2).wait()

        @plsc.parallel_loop(0, _C, step=16, unroll=8)
        def body(i):
            ix = pl.ds(i, 16)
            ov[p, ix] = _sample(lv[p, ix], av[p, ix], bv[p, ix])

        out_h[c] = pltpu.async_copy(
            ov.at[p], out_hbm.at[pl.ds(base + c * _C, _C)], osems[p]
        )
    for c in sorted(out_h):
        out_h.pop(c).wait()


@functools.cache
def _sc_call():
    return pl.kernel(
        _sc_body,
        out_type=jax.ShapeDtypeStruct((_N,), jnp.float32),
        mesh=plsc.VectorSubcoreMesh(core_axis_name="c", subcore_axis_name="s"),
        scratch_types=[
            pltpu.VMEM((2, _C), jnp.float32),
            pltpu.VMEM((2, _C), jnp.float32),
            pltpu.VMEM((2, _C), jnp.float32),
            pltpu.VMEM((2, _C), jnp.float32),
            pltpu.SemaphoreType.DMA,
            pltpu.SemaphoreType.DMA,
            pltpu.SemaphoreType.DMA,
            pltpu.SemaphoreType.DMA,
        ],
    )


@jax.jit
def kernel(logits, uniform_a, uniform_b):
    l = logits.reshape(_N)
    ua = uniform_a.reshape(_N)
    ub = uniform_b.reshape(_N)
    out = _sc_call()(l, ua, ub)
    return out.reshape(_B, _S, 1)
